# initial kernel scaffold (unmeasured)
import jax
import jax.numpy as jnp
from jax import lax
from jax.experimental import pallas as pl
from jax.experimental.pallas import tpu as pltpu

T = 2048
D = 1024
CH = 128
NCH = T // CH


def kernel(x, dest):
    my_y = lax.axis_index("y")

    order = jnp.argsort(dest, stable=True)
    xs = x[order, :].astype(jnp.bfloat16)

    n0 = jnp.sum((dest == 0).astype(jnp.int32))
    n_keep = jnp.where(my_y == 0, n0, T - n0).astype(jnp.int32)
    m = (T - n_keep).astype(jnp.int32)
    keep_off = my_y * m
    send_src = (1 - my_y) * n_keep
    remote_dst = my_y * n_keep
    nch_m = (m + CH - 1) // CH
    nch_k = (n_keep + CH - 1) // CH
    meta = jnp.stack(
        [n_keep, m, keep_off, send_src, remote_dst, nch_m, nch_k]
    ).astype(jnp.int32)

    def body(meta_ref, xs_ref, out_ref, send_sems, recv_sems):
        n_keep_ = meta_ref[0]
        m_ = meta_ref[1]
        keep_off_ = meta_ref[2]
        send_src_ = meta_ref[3]
        remote_dst_ = meta_ref[4]
        nch_m_ = meta_ref[5]
        nch_k_ = meta_ref[6]

        ax = lax.axis_index("x")
        ay = lax.axis_index("y")
        az = lax.axis_index("z")
        peer = (ax, 1 - ay, az)

        barrier = pltpu.get_barrier_semaphore()
        pl.semaphore_signal(
            barrier, inc=1, device_id=peer, device_id_type=pl.DeviceIdType.MESH
        )
        pl.semaphore_wait(barrier, 1)

        def cstart(i, n):
            return jnp.maximum(0, jnp.minimum(i * CH, n - CH))

        def mk_chunk(i):
            s = cstart(i, m_)
            return pltpu.make_async_remote_copy(
                src_ref=xs_ref.at[pl.ds(send_src_ + s, CH), :],
                dst_ref=out_ref.at[pl.ds(remote_dst_ + s, CH), :],
                send_sem=send_sems.at[i],
                recv_sem=recv_sems.at[i],
                device_id=peer,
                device_id_type=pl.DeviceIdType.MESH,
            )

        for i in range(NCH):
            rdma = mk_chunk(i)

            @pl.when(i < nch_m_)
            def _(rdma=rdma):
                rdma.start()

        for i in range(NCH):

            @pl.when(i < nch_k_)
            def _(i=i):
                s = cstart(i, n_keep_)
                out_ref[pl.ds(keep_off_ + s, CH), :] = xs_ref[
                    pl.ds(keep_off_ + s, CH), :
                ]

        for i in range(NCH):
            rdma = mk_chunk(i)

            @pl.when(i < nch_m_)
            def _(rdma=rdma):
                rdma.wait_send()
                rdma.wait_recv()

    return pl.pallas_call(
        body,
        out_shape=jax.ShapeDtypeStruct((T, D), jnp.bfloat16),
        in_specs=[
            pl.BlockSpec(memory_space=pltpu.SMEM),
            pl.BlockSpec(memory_space=pltpu.VMEM),
        ],
        out_specs=pl.BlockSpec(memory_space=pltpu.VMEM),
        scratch_shapes=[
            pltpu.SemaphoreType.DMA((NCH,)),
            pltpu.SemaphoreType.DMA((NCH,)),
        ],
        compiler_params=pltpu.CompilerParams(collective_id=0),
    )(meta, xs)


# baseline (device time: 62041 ns/iter reference)
import jax
import jax.numpy as jnp
from jax import lax
from jax.experimental import pallas as pl
from jax.experimental.pallas import tpu as pltpu

T = 2048
D = 1024
CH = 128
NCH = T // CH


def kernel(x, dest):
    my_y = lax.axis_index("y")

    order = jnp.argsort(dest, stable=True)
    xs = x[order, :].astype(jnp.bfloat16)

    n0 = jnp.sum((dest == 0).astype(jnp.int32))
    n_keep = jnp.where(my_y == 0, n0, T - n0).astype(jnp.int32)
    m = (T - n_keep).astype(jnp.int32)
    keep_off = my_y * m
    send_src = (1 - my_y) * n_keep
    recv_dst = (1 - my_y) * n_keep
    nch = (m + CH - 1) // CH

    send_buf = jnp.roll(xs, -send_src, axis=0)

    meta = jnp.stack([m, nch]).astype(jnp.int32)

    def body(meta_ref, sbuf_ref, recv_ref, send_sems, recv_sems):
        m_ = meta_ref[0]
        nch_ = meta_ref[1]

        ax = lax.axis_index("x")
        ay = lax.axis_index("y")
        az = lax.axis_index("z")
        peer = (ax, 1 - ay, az)

        barrier = pltpu.get_barrier_semaphore()
        pl.semaphore_signal(
            barrier, inc=1, device_id=peer, device_id_type=pl.DeviceIdType.MESH
        )
        pl.semaphore_wait(barrier, 1)

        def cstart(i):
            tail = jnp.maximum(0, ((m_ + 7) // 8) * 8 - CH)
            s = jnp.where(i == nch_ - 1, tail, i * CH)
            return pl.multiple_of(s, 8)

        def mk_chunk(i):
            s = cstart(i)
            return pltpu.make_async_remote_copy(
                src_ref=sbuf_ref.at[pl.ds(s, CH), :],
                dst_ref=recv_ref.at[pl.ds(s, CH), :],
                send_sem=send_sems.at[i],
                recv_sem=recv_sems.at[i],
                device_id=peer,
                device_id_type=pl.DeviceIdType.MESH,
            )

        for i in range(NCH):
            rdma = mk_chunk(i)

            @pl.when(i < nch_)
            def _(rdma=rdma):
                rdma.start()

        for i in range(NCH):
            rdma = mk_chunk(i)

            @pl.when(i < nch_)
            def _(rdma=rdma):
                rdma.wait_send()
                rdma.wait_recv()

    recv = pl.pallas_call(
        body,
        out_shape=jax.ShapeDtypeStruct((T, D), jnp.bfloat16),
        in_specs=[
            pl.BlockSpec(memory_space=pltpu.SMEM),
            pl.BlockSpec(memory_space=pltpu.VMEM),
        ],
        out_specs=pl.BlockSpec(memory_space=pltpu.VMEM),
        scratch_shapes=[
            pltpu.SemaphoreType.DMA((NCH,)),
            pltpu.SemaphoreType.DMA((NCH,)),
        ],
        compiler_params=pltpu.CompilerParams(collective_id=0),
    )(meta, send_buf)

    r_idx = jnp.arange(T, dtype=jnp.int32)
    mine = (r_idx >= keep_off) & (r_idx < keep_off + n_keep)
    peer_rows = recv[jnp.clip(r_idx - recv_dst, 0, T - 1), :]
    return jnp.where(mine[:, None], xs, peer_rows)
